# Initial kernel scaffold; baseline (speedup 1.0000x reference)
import jax
import jax.numpy as jnp
from jax.experimental import pallas as pl


def kernel(context, scene_probs, W1, b1, W2, b2, scene_priors, prior_weight):
    raise NotImplementedError("write your pallas kernel here")


# speedup vs baseline: 17.6316x; 17.6316x over previous
import functools

import jax
import jax.numpy as jnp
from jax.experimental import pallas as pl
from jax.experimental.pallas import tpu as pltpu

B = 16384
CTX = 256
HID = 64
M = 1024
TOP_K = 8
NS_PAD = 8

ROWS = 512


def _router_kernel(ctx_ref, sp_ref, w1_ref, b1_ref, w2_ref, b2_ref,
                   priors_ref, pw_ref, sel_ref, comb_ref):
    ctx = ctx_ref[...]
    h = jnp.maximum(
        jax.lax.dot_general(ctx, w1_ref[...], (((1,), (0,)), ((), ())),
                            preferred_element_type=jnp.float32) + b1_ref[...],
        0.0)
    logits = jax.lax.dot_general(h, w2_ref[...], (((1,), (0,)), ((), ())),
                                 preferred_element_type=jnp.float32) + b2_ref[...]
    content_probs = jax.nn.sigmoid(logits)

    priors = jax.nn.sigmoid(priors_ref[...])
    scene_prior = jax.lax.dot_general(
        sp_ref[...], priors, (((1,), (0,)), ((), ())),
        preferred_element_type=jnp.float32)

    alpha = jax.nn.sigmoid(pw_ref[0, 0])
    combined = alpha * scene_prior + (1.0 - alpha) * content_probs

    t = combined
    kth = None
    for _ in range(TOP_K):
        kth = jnp.max(t, axis=1, keepdims=True)
        t = jnp.where(t >= kth, -jnp.inf, t)
    mask = (combined >= kth).astype(jnp.float32)

    sel_ref[...] = 0.9 * mask + 0.1 * combined
    comb_ref[...] = combined


@jax.jit
def _run(context, scene_probs_p, W1, b1, W2, b2, priors_p, pw):
    grid = (B // ROWS,)
    full = lambda i: (0, 0)
    row_blk = lambda i: (i, 0)
    out_shape = jax.ShapeDtypeStruct((B, M), jnp.float32)
    sel, comb = pl.pallas_call(
        _router_kernel,
        grid=grid,
        in_specs=[
            pl.BlockSpec((ROWS, CTX), row_blk),
            pl.BlockSpec((ROWS, NS_PAD), row_blk),
            pl.BlockSpec((CTX, HID), full),
            pl.BlockSpec((1, HID), full),
            pl.BlockSpec((HID, M), full),
            pl.BlockSpec((1, M), full),
            pl.BlockSpec((NS_PAD, M), full),
            pl.BlockSpec(memory_space=pltpu.SMEM),
        ],
        out_specs=[pl.BlockSpec((ROWS, M), row_blk),
                   pl.BlockSpec((ROWS, M), row_blk)],
        out_shape=[out_shape, out_shape],
        compiler_params=pltpu.CompilerParams(
            dimension_semantics=("arbitrary",),
        ),
    )(context, scene_probs_p, W1, b1, W2, b2, priors_p, pw)
    return sel, comb


def kernel(context, scene_probs, W1, b1, W2, b2, scene_priors, prior_weight):
    ns = scene_probs.shape[1]
    scene_probs_p = jnp.pad(scene_probs, ((0, 0), (0, NS_PAD - ns)))
    priors_p = jnp.pad(scene_priors, ((0, NS_PAD - ns), (0, 0)),
                       constant_values=-jnp.inf)
    pw = jnp.reshape(prior_weight, (1, 1))
    return _run(context, scene_probs_p, W1, jnp.reshape(b1, (1, HID)),
                W2, jnp.reshape(b2, (1, M)), priors_p, pw)


# speedup vs baseline: 17.7295x; 1.0056x over previous
import functools

import jax
import jax.numpy as jnp
from jax.experimental import pallas as pl
from jax.experimental.pallas import tpu as pltpu

B = 16384
CTX = 256
HID = 64
M = 1024
TOP_K = 8
NS_PAD = 8

ROWS = 1024


def _router_kernel(ctx_ref, sp_ref, w1_ref, b1_ref, w2_ref, b2_ref,
                   priors_ref, pw_ref, sel_ref, comb_ref):
    ctx = ctx_ref[...]
    h = jnp.maximum(
        jax.lax.dot_general(ctx, w1_ref[...], (((1,), (0,)), ((), ())),
                            preferred_element_type=jnp.float32) + b1_ref[...],
        0.0)
    logits = jax.lax.dot_general(h, w2_ref[...], (((1,), (0,)), ((), ())),
                                 preferred_element_type=jnp.float32) + b2_ref[...]
    content_probs = jax.nn.sigmoid(logits)

    priors = jax.nn.sigmoid(priors_ref[...])
    scene_prior = jax.lax.dot_general(
        sp_ref[...], priors, (((1,), (0,)), ((), ())),
        preferred_element_type=jnp.float32)

    alpha = jax.nn.sigmoid(pw_ref[0, 0])
    combined = alpha * scene_prior + (1.0 - alpha) * content_probs

    t = combined
    kth = None
    for _ in range(TOP_K):
        kth = jnp.max(t, axis=1, keepdims=True)
        t = jnp.where(t >= kth, -jnp.inf, t)
    mask = (combined >= kth).astype(jnp.float32)

    sel_ref[...] = 0.9 * mask + 0.1 * combined
    comb_ref[...] = combined


@jax.jit
def _run(context, scene_probs_p, W1, b1, W2, b2, priors_p, pw):
    grid = (B // ROWS,)
    full = lambda i: (0, 0)
    row_blk = lambda i: (i, 0)
    out_shape = jax.ShapeDtypeStruct((B, M), jnp.float32)
    sel, comb = pl.pallas_call(
        _router_kernel,
        grid=grid,
        in_specs=[
            pl.BlockSpec((ROWS, CTX), row_blk),
            pl.BlockSpec((ROWS, NS_PAD), row_blk),
            pl.BlockSpec((CTX, HID), full),
            pl.BlockSpec((1, HID), full),
            pl.BlockSpec((HID, M), full),
            pl.BlockSpec((1, M), full),
            pl.BlockSpec((NS_PAD, M), full),
            pl.BlockSpec(memory_space=pltpu.SMEM),
        ],
        out_specs=[pl.BlockSpec((ROWS, M), row_blk),
                   pl.BlockSpec((ROWS, M), row_blk)],
        out_shape=[out_shape, out_shape],
        compiler_params=pltpu.CompilerParams(
            dimension_semantics=("arbitrary",),
        ),
    )(context, scene_probs_p, W1, b1, W2, b2, priors_p, pw)
    return sel, comb


def kernel(context, scene_probs, W1, b1, W2, b2, scene_priors, prior_weight):
    ns = scene_probs.shape[1]
    scene_probs_p = jnp.pad(scene_probs, ((0, 0), (0, NS_PAD - ns)))
    priors_p = jnp.pad(scene_priors, ((0, NS_PAD - ns), (0, 0)),
                       constant_values=-jnp.inf)
    pw = jnp.reshape(prior_weight, (1, 1))
    return _run(context, scene_probs_p, W1, jnp.reshape(b1, (1, HID)),
                W2, jnp.reshape(b2, (1, M)), priors_p, pw)
